# baseline (speedup 1.0000x reference)
import functools

import jax
import jax.numpy as jnp
from jax import lax
from jax.experimental import pallas as pl
from jax.experimental.pallas import tpu as pltpu
from jax.experimental.pallas import tpu_sc as plsc

_NC, _NS = 2, 16
_NW = _NC * _NS


def _tc_meta(tki_t, hidden_states, num_experts, bm, nblk, t, k):
    n = t * k
    nseg = 32
    nrow = n // nseg
    ne = num_experts
    nc = nseg * ne
    h = hidden_states.shape[1]
    xblk = 512
    mrows = ((nblk + 1 + 7) // 8) * 8

    def outer_body(tki_ref, x_ref, gd_ref, meta_ref, xbf_ref):
        xbf_ref[...] = x_ref[...].astype(jnp.bfloat16)

        @pl.when(pl.program_id(0) == 0)
        def _():
            meta_body(tki_ref, gd_ref, meta_ref)

    def meta_body(tki_ref, gd_ref, meta_ref):
        fdot = functools.partial(jnp.dot, preferred_element_type=jnp.float32)

        def iot(shape, dim):
            return lax.broadcasted_iota(jnp.int32, shape, dim)

        ev_f = tki_ref[...].astype(jnp.float32)
        rep = (iot((nseg, nc), 1) // ne == iot((nseg, nc), 0)).astype(jnp.float32)
        ev_rep = fdot(ev_f, rep)
        ecol = (iot((nrow, nc), 1) % ne).astype(jnp.float32)
        oh = (ev_rep == ecol).astype(jnp.float32)
        tri = (iot((nrow, nrow), 0) >= iot((nrow, nrow), 1)).astype(jnp.float32)
        within = fdot(tri, oh)
        seg_flat = jnp.sum(oh, axis=0, keepdims=True)
        segm = ((iot((nc, nc), 0) // ne < iot((nc, nc), 1) // ne)
                & (iot((nc, nc), 0) % ne == iot((nc, nc), 1) % ne)
                ).astype(jnp.float32)
        prior = fdot(seg_flat, segm)
        rank_incl = within + prior
        csel = (iot((nc, ne), 0) % ne == iot((nc, ne), 1)).astype(jnp.float32)
        counts = fdot(seg_flat, csel)
        nblk_f = jnp.floor((counts + (bm - 1)) * (1.0 / bm))
        upper = (iot((ne, ne), 0) < iot((ne, ne), 1)).astype(jnp.float32)
        start_f = fdot(nblk_f, upper)
        start_rep = fdot(start_f, (iot((ne, nc), 1) % ne
                                   == iot((ne, nc), 0)).astype(jnp.float32))
        sel = oh * (start_rep * bm + rank_incl - 1.0)
        gsel = (iot((nc, nseg), 0) // ne == iot((nc, nseg), 1)).astype(jnp.float32)
        gd_ref[...] = jnp.dot(sel, gsel, preferred_element_type=jnp.float32,
                              precision=lax.Precision.HIGHEST).astype(jnp.int32)
        num_active = jnp.sum(nblk_f).astype(jnp.int32)
        start_i = start_f.astype(jnp.int32)
        br = lax.broadcasted_iota(jnp.int32, (mrows, ne), 0)
        be = jnp.sum((start_i <= br).astype(jnp.int32),
                     axis=1, keepdims=True) - 1
        be_last = jnp.sum((start_i <= num_active - 1).astype(jnp.int32)) - 1
        bcol = lax.broadcasted_iota(jnp.int32, (mrows, 1), 0)
        be = jnp.where(bcol < num_active, be, be_last)
        meta_ref[...] = jnp.where(bcol == nblk, num_active, be)

    return pl.pallas_call(
        outer_body,
        grid=(t // xblk,),
        in_specs=[pl.BlockSpec((nrow, nseg), lambda i: (0, 0)),
                  pl.BlockSpec((xblk, h), lambda i: (i, 0))],
        out_specs=[pl.BlockSpec((nrow, nseg), lambda i: (0, 0)),
                   pl.BlockSpec((mrows, 1), lambda i: (0, 0)),
                   pl.BlockSpec((xblk, h), lambda i: (i, 0))],
        out_shape=[jax.ShapeDtypeStruct((nrow, nseg), jnp.int32),
                   jax.ShapeDtypeStruct((mrows, 1), jnp.int32),
                   jax.ShapeDtypeStruct((t, h), jnp.bfloat16)],
    )(tki_t.reshape(nrow, nseg), hidden_states)


def _tc_ffn(x_bf, gd, wd, meta, gate_up_proj, down_proj, bm, nblk, pad):
    e, h, i2 = gate_up_proj.shape
    i = i2 // 2
    t = x_bf.shape[0]

    def body(meta_ref, gd_ref, wd_ref, x_ref, gu_ref, dp_ref, out_ref):
        b = pl.program_id(0)

        @pl.when(b < meta_ref[nblk])
        def _():
            rows = lax.broadcasted_iota(jnp.int32, (bm, t), 0) + b * bm
            cmp0 = gd_ref[0:1, :] == rows
            cmp1 = gd_ref[1:2, :] == rows
            onehot = cmp0.astype(jnp.bfloat16) + cmp1.astype(jnp.bfloat16)
            wrow = jnp.sum(
                jnp.where(cmp0, wd_ref[0:1, :], 0.0)
                + jnp.where(cmp1, wd_ref[1:2, :], 0.0),
                axis=1, keepdims=True)
            x = jnp.dot(onehot, x_ref[...],
                        preferred_element_type=jnp.float32).astype(jnp.bfloat16)
            gu = jnp.dot(x, gu_ref[0].astype(jnp.bfloat16),
                         preferred_element_type=jnp.float32)
            gate = gu[:, :i]
            up = gu[:, i:]
            act = gate * jax.nn.sigmoid(gate) * up * wrow
            out_ref[...] = jnp.dot(act.astype(jnp.bfloat16),
                                   dp_ref[0].astype(jnp.bfloat16),
                                   preferred_element_type=jnp.float32)

    grid_spec = pltpu.PrefetchScalarGridSpec(
        num_scalar_prefetch=1,
        grid=(nblk,),
        in_specs=[
            pl.BlockSpec((2, t), lambda b, m: (0, 0)),
            pl.BlockSpec((2, t), lambda b, m: (0, 0)),
            pl.BlockSpec((t, h), lambda b, m: (0, 0)),
            pl.BlockSpec((1, h, i2), lambda b, m: (m[b], 0, 0)),
            pl.BlockSpec((1, i, h), lambda b, m: (m[b], 0, 0)),
        ],
        out_specs=pl.BlockSpec((bm, h), lambda b, m: (b, 0)),
    )
    return pl.pallas_call(
        body,
        grid_spec=grid_spec,
        out_shape=jax.ShapeDtypeStruct((pad, h), jnp.float32),
    )(meta, gd, wd, x_bf, gate_up_proj, down_proj)


def _sc_combine(part, g0, g1, t, h):
    tpw = t // _NW
    ch = 16
    nch = tpw // ch
    g0 = g0.reshape(_NW, nch, ch)
    g1 = g1.reshape(_NW, nch, ch)
    mesh = plsc.VectorSubcoreMesh(core_axis_name="c", subcore_axis_name="s")
    nvec = ch * (h // 16)
    cshift = 0
    hh = h // 16
    while (1 << cshift) < hh:
        cshift += 1

    @functools.partial(
        pl.kernel, mesh=mesh,
        out_type=jax.ShapeDtypeStruct((t, h), jnp.float32),
        scratch_types=[
            pltpu.VMEM((nch, ch), jnp.int32),
            pltpu.VMEM((nch, ch), jnp.int32),
            pltpu.VMEM((ch, h), jnp.float32),
            pltpu.VMEM((ch, h), jnp.float32),
            pltpu.SemaphoreType.DMA,
            pltpu.SemaphoreType.DMA,
        ],
    )
    def k(part_hbm, g0_hbm, g1_hbm, out_hbm, i0, i1, ba, bb, sa, sb):
        wid = lax.axis_index("s") * _NC + lax.axis_index("c")
        base = wid * tpw
        pltpu.sync_copy(g0_hbm.at[wid], i0)
        pltpu.sync_copy(g1_hbm.at[wid], i1)
        for j in range(nch):
            ca = pltpu.async_copy(part_hbm.at[i0.at[j]], ba, sa)
            cb = pltpu.async_copy(part_hbm.at[i1.at[j]], bb, sb)
            ca.wait()
            cb.wait()

            def add_body(tt, carry):
                r = lax.shift_right_logical(tt, cshift)
                c = pl.multiple_of(lax.shift_left(lax.bitwise_and(tt, hh - 1), 4), 16)
                ba[r, pl.ds(c, 16)] = ba[r, pl.ds(c, 16)] + bb[r, pl.ds(c, 16)]
                return carry

            lax.fori_loop(0, nvec, add_body, 0, unroll=4)
            pltpu.sync_copy(ba, out_hbm.at[pl.ds(base + j * ch, ch)])

    return k(part, g0, g1)


def kernel(hidden_states, top_k_index, top_k_weights, gate_up_proj, down_proj):
    t, h = hidden_states.shape
    e = gate_up_proj.shape[0]
    k = top_k_index.shape[1]
    bm = 128
    n = t * k
    nblk = n // bm + e
    pad = nblk * bm

    tki_t = top_k_index.T.astype(jnp.int32)
    wd = top_k_weights.T.astype(jnp.float32)
    gd128, meta2, x_bf = _tc_meta(tki_t, hidden_states, e, bm, nblk, t, k)
    gd = gd128.reshape(k, t)
    meta = meta2.reshape(meta2.shape[0])
    part = _tc_ffn(x_bf, gd, wd, meta, gate_up_proj, down_proj, bm, nblk, pad)
    return _sc_combine(part, gd[0], gd[1], t, h)


# speedup vs baseline: 1.0636x; 1.0636x over previous
import functools

import jax
import jax.numpy as jnp
from jax import lax
from jax.experimental import pallas as pl
from jax.experimental.pallas import tpu as pltpu
from jax.experimental.pallas import tpu_sc as plsc

_NC, _NS = 2, 16
_NW = _NC * _NS


def _tc_meta(tki_t, hidden_states, num_experts, bm, nblk, t, k):
    n = t * k
    nseg = 32
    nrow = n // nseg
    ne = num_experts
    nc = nseg * ne
    h = hidden_states.shape[1]
    xblk = 512
    mrows = ((nblk + 1 + 7) // 8) * 8

    def outer_body(tki_ref, x_ref, gd_ref, meta_ref, xbf_ref):
        xbf_ref[...] = x_ref[...].astype(jnp.bfloat16)

        @pl.when(pl.program_id(0) == 0)
        def _():
            meta_body(tki_ref, gd_ref, meta_ref)

    def meta_body(tki_ref, gd_ref, meta_ref):
        fdot = functools.partial(jnp.dot, preferred_element_type=jnp.float32)

        def iot(shape, dim):
            return lax.broadcasted_iota(jnp.int32, shape, dim)

        ev_f = tki_ref[...].astype(jnp.float32)
        rep = (iot((nseg, nc), 1) // ne == iot((nseg, nc), 0)).astype(jnp.float32)
        ev_rep = fdot(ev_f, rep)
        ecol = (iot((nrow, nc), 1) % ne).astype(jnp.float32)
        oh = (ev_rep == ecol).astype(jnp.float32)
        tri = (iot((nrow, nrow), 0) >= iot((nrow, nrow), 1)).astype(jnp.float32)
        within = fdot(tri, oh)
        seg_flat = jnp.sum(oh, axis=0, keepdims=True)
        segm = ((iot((nc, nc), 0) // ne < iot((nc, nc), 1) // ne)
                & (iot((nc, nc), 0) % ne == iot((nc, nc), 1) % ne)
                ).astype(jnp.float32)
        prior = fdot(seg_flat, segm)
        rank_incl = within + prior
        csel = (iot((nc, ne), 0) % ne == iot((nc, ne), 1)).astype(jnp.float32)
        counts = fdot(seg_flat, csel)
        nblk_f = jnp.floor((counts + (bm - 1)) * (1.0 / bm))
        upper = (iot((ne, ne), 0) < iot((ne, ne), 1)).astype(jnp.float32)
        start_f = fdot(nblk_f, upper)
        start_rep = fdot(start_f, (iot((ne, nc), 1) % ne
                                   == iot((ne, nc), 0)).astype(jnp.float32))
        sel = oh * (start_rep * bm + rank_incl - 1.0)
        gsel = (iot((nc, nseg), 0) // ne == iot((nc, nseg), 1)).astype(jnp.float32)
        gd_ref[...] = jnp.dot(sel, gsel, preferred_element_type=jnp.float32,
                              precision=lax.Precision.HIGHEST).astype(jnp.int32)
        num_active = jnp.sum(nblk_f).astype(jnp.int32)
        start_i = start_f.astype(jnp.int32)
        br = lax.broadcasted_iota(jnp.int32, (mrows, ne), 0)
        be = jnp.sum((start_i <= br).astype(jnp.int32),
                     axis=1, keepdims=True) - 1
        be_last = jnp.sum((start_i <= num_active - 1).astype(jnp.int32)) - 1
        bcol = lax.broadcasted_iota(jnp.int32, (mrows, 1), 0)
        be = jnp.where(bcol < num_active, be, be_last)
        meta_ref[...] = jnp.where(bcol == nblk, num_active, be)

    return pl.pallas_call(
        outer_body,
        grid=(t // xblk,),
        in_specs=[pl.BlockSpec((nrow, nseg), lambda i: (0, 0)),
                  pl.BlockSpec((xblk, h), lambda i: (i, 0))],
        out_specs=[pl.BlockSpec((nrow, nseg), lambda i: (0, 0)),
                   pl.BlockSpec((mrows, 1), lambda i: (0, 0)),
                   pl.BlockSpec((xblk, h), lambda i: (i, 0))],
        out_shape=[jax.ShapeDtypeStruct((nrow, nseg), jnp.int32),
                   jax.ShapeDtypeStruct((mrows, 1), jnp.int32),
                   jax.ShapeDtypeStruct((t, h), jnp.bfloat16)],
    )(tki_t.reshape(nrow, nseg), hidden_states)


def _tc_ffn(x_bf, gd, wd, meta, gate_up_proj, down_proj, bm, nblk, pad):
    e, h, i2 = gate_up_proj.shape
    i = i2 // 2
    t = x_bf.shape[0]

    def body(meta_ref, gd_ref, wd_ref, x_ref, gu_ref, dp_ref, out_ref):
        b = pl.program_id(0)

        @pl.when(b < meta_ref[nblk])
        def _():
            rows = lax.broadcasted_iota(jnp.int32, (bm, t), 0) + b * bm
            cmp0 = gd_ref[0:1, :] == rows
            cmp1 = gd_ref[1:2, :] == rows
            onehot = (cmp0 | cmp1).astype(jnp.bfloat16)
            wrow = jnp.sum(
                jnp.where(cmp0, wd_ref[0:1, :],
                          jnp.where(cmp1, wd_ref[1:2, :], 0.0)),
                axis=1, keepdims=True)
            x = jnp.dot(onehot, x_ref[...],
                        preferred_element_type=jnp.float32).astype(jnp.bfloat16)
            gu = jnp.dot(x, gu_ref[0].astype(jnp.bfloat16),
                         preferred_element_type=jnp.float32)
            gate = gu[:, :i]
            up = gu[:, i:]
            act = gate * jax.nn.sigmoid(gate) * up * wrow
            out_ref[...] = jnp.dot(act.astype(jnp.bfloat16),
                                   dp_ref[0].astype(jnp.bfloat16),
                                   preferred_element_type=jnp.float32)

    grid_spec = pltpu.PrefetchScalarGridSpec(
        num_scalar_prefetch=1,
        grid=(nblk,),
        in_specs=[
            pl.BlockSpec((2, t), lambda b, m: (0, 0)),
            pl.BlockSpec((2, t), lambda b, m: (0, 0)),
            pl.BlockSpec((t, h), lambda b, m: (0, 0)),
            pl.BlockSpec((1, h, i2), lambda b, m: (m[b], 0, 0)),
            pl.BlockSpec((1, i, h), lambda b, m: (m[b], 0, 0)),
        ],
        out_specs=pl.BlockSpec((bm, h), lambda b, m: (b, 0)),
    )
    return pl.pallas_call(
        body,
        grid_spec=grid_spec,
        out_shape=jax.ShapeDtypeStruct((pad, h), jnp.float32),
    )(meta, gd, wd, x_bf, gate_up_proj, down_proj)


def _sc_combine(part, g0, g1, t, h):
    tpw = t // _NW
    ch = 16
    nch = tpw // ch
    g0 = g0.reshape(_NW, nch, ch)
    g1 = g1.reshape(_NW, nch, ch)
    mesh = plsc.VectorSubcoreMesh(core_axis_name="c", subcore_axis_name="s")
    nvec = ch * (h // 16)
    cshift = 0
    hh = h // 16
    while (1 << cshift) < hh:
        cshift += 1

    @functools.partial(
        pl.kernel, mesh=mesh,
        out_type=jax.ShapeDtypeStruct((t, h), jnp.float32),
        scratch_types=[
            pltpu.VMEM((nch, ch), jnp.int32),
            pltpu.VMEM((nch, ch), jnp.int32),
            pltpu.VMEM((ch, h), jnp.float32),
            pltpu.VMEM((ch, h), jnp.float32),
            pltpu.SemaphoreType.DMA,
            pltpu.SemaphoreType.DMA,
        ],
    )
    def k(part_hbm, g0_hbm, g1_hbm, out_hbm, i0, i1, ba, bb, sa, sb):
        wid = lax.axis_index("s") * _NC + lax.axis_index("c")
        base = wid * tpw
        pltpu.sync_copy(g0_hbm.at[wid], i0)
        pltpu.sync_copy(g1_hbm.at[wid], i1)
        for j in range(nch):
            ca = pltpu.async_copy(part_hbm.at[i0.at[j]], ba, sa)
            cb = pltpu.async_copy(part_hbm.at[i1.at[j]], bb, sb)
            ca.wait()
            cb.wait()

            def add_body(tt, carry):
                r = lax.shift_right_logical(tt, cshift)
                c = pl.multiple_of(lax.shift_left(lax.bitwise_and(tt, hh - 1), 4), 16)
                ba[r, pl.ds(c, 16)] = ba[r, pl.ds(c, 16)] + bb[r, pl.ds(c, 16)]
                return carry

            lax.fori_loop(0, nvec, add_body, 0, unroll=4)
            pltpu.sync_copy(ba, out_hbm.at[pl.ds(base + j * ch, ch)])

    return k(part, g0, g1)


def kernel(hidden_states, top_k_index, top_k_weights, gate_up_proj, down_proj):
    t, h = hidden_states.shape
    e = gate_up_proj.shape[0]
    k = top_k_index.shape[1]
    bm = 256
    n = t * k
    nblk = n // bm + e
    pad = nblk * bm

    tki_t = top_k_index.T.astype(jnp.int32)
    wd = top_k_weights.T.astype(jnp.float32)
    gd128, meta2, x_bf = _tc_meta(tki_t, hidden_states, e, bm, nblk, t, k)
    gd = gd128.reshape(k, t)
    meta = meta2.reshape(meta2.shape[0])
    part = _tc_ffn(x_bf, gd, wd, meta, gate_up_proj, down_proj, bm, nblk, pad)
    return _sc_combine(part, gd[0], gd[1], t, h)


# speedup vs baseline: 1.1104x; 1.0440x over previous
import functools

import jax
import jax.numpy as jnp
from jax import lax
from jax.experimental import pallas as pl
from jax.experimental.pallas import tpu as pltpu
from jax.experimental.pallas import tpu_sc as plsc

_NC, _NS = 2, 16
_NW = _NC * _NS


def _tc_meta(tki_t, hidden_states, num_experts, bm, nblk, t, k):
    n = t * k
    nseg = 32
    nrow = n // nseg
    ne = num_experts
    nc = nseg * ne
    h = hidden_states.shape[1]
    xblk = 512
    mrows = ((nblk + 1 + 7) // 8) * 8

    def outer_body(tki_ref, x_ref, gd_ref, meta_ref, xbf_ref):
        xbf_ref[...] = x_ref[...].astype(jnp.bfloat16)

        @pl.when(pl.program_id(0) == 0)
        def _():
            meta_body(tki_ref, gd_ref, meta_ref)

    def meta_body(tki_ref, gd_ref, meta_ref):
        fdot = functools.partial(jnp.dot, preferred_element_type=jnp.float32)

        def iot(shape, dim):
            return lax.broadcasted_iota(jnp.int32, shape, dim)

        ev_f = tki_ref[...].astype(jnp.float32)
        rep = (iot((nseg, nc), 1) // ne == iot((nseg, nc), 0)).astype(jnp.float32)
        ev_rep = fdot(ev_f, rep)
        ecol = (iot((nrow, nc), 1) % ne).astype(jnp.float32)
        oh = (ev_rep == ecol).astype(jnp.float32)
        tri = (iot((nrow, nrow), 0) >= iot((nrow, nrow), 1)).astype(jnp.float32)
        within = fdot(tri, oh)
        seg_flat = jnp.sum(oh, axis=0, keepdims=True)
        segm = ((iot((nc, nc), 0) // ne < iot((nc, nc), 1) // ne)
                & (iot((nc, nc), 0) % ne == iot((nc, nc), 1) % ne)
                ).astype(jnp.float32)
        prior = fdot(seg_flat, segm)
        rank_incl = within + prior
        csel = (iot((nc, ne), 0) % ne == iot((nc, ne), 1)).astype(jnp.float32)
        counts = fdot(seg_flat, csel)
        nblk_f = jnp.floor((counts + (bm - 1)) * (1.0 / bm))
        upper = (iot((ne, ne), 0) < iot((ne, ne), 1)).astype(jnp.float32)
        start_f = fdot(nblk_f, upper)
        start_rep = fdot(start_f, (iot((ne, nc), 1) % ne
                                   == iot((ne, nc), 0)).astype(jnp.float32))
        sel = oh * (start_rep * bm + rank_incl - 1.0)
        gsel = (iot((nc, nseg), 0) // ne == iot((nc, nseg), 1)).astype(jnp.float32)
        gd_ref[...] = jnp.dot(sel, gsel, preferred_element_type=jnp.float32,
                              precision=lax.Precision.HIGHEST).astype(jnp.int32)
        num_active = jnp.sum(nblk_f).astype(jnp.int32)
        start_i = start_f.astype(jnp.int32)
        br = lax.broadcasted_iota(jnp.int32, (mrows, ne), 0)
        be = jnp.sum((start_i <= br).astype(jnp.int32),
                     axis=1, keepdims=True) - 1
        be_last = jnp.sum((start_i <= num_active - 1).astype(jnp.int32)) - 1
        bcol = lax.broadcasted_iota(jnp.int32, (mrows, 1), 0)
        be = jnp.where(bcol < num_active, be, be_last)
        meta_ref[...] = jnp.where(bcol == nblk, num_active, be)

    return pl.pallas_call(
        outer_body,
        grid=(t // xblk,),
        in_specs=[pl.BlockSpec((nrow, nseg), lambda i: (0, 0)),
                  pl.BlockSpec((xblk, h), lambda i: (i, 0))],
        out_specs=[pl.BlockSpec((nrow, nseg), lambda i: (0, 0)),
                   pl.BlockSpec((mrows, 1), lambda i: (0, 0)),
                   pl.BlockSpec((xblk, h), lambda i: (i, 0))],
        out_shape=[jax.ShapeDtypeStruct((nrow, nseg), jnp.int32),
                   jax.ShapeDtypeStruct((mrows, 1), jnp.int32),
                   jax.ShapeDtypeStruct((t, h), jnp.bfloat16)],
    )(tki_t.reshape(nrow, nseg), hidden_states)


def _tc_ffn(x_bf, gd, wd, meta, gate_up_proj, down_proj, bm, nblk, pad):
    e, h, i2 = gate_up_proj.shape
    i = i2 // 2
    t = x_bf.shape[0]

    def body(meta_ref, gd_ref, wd_ref, x_ref, gu_ref, dp_ref, out_ref):
        b = pl.program_id(0)

        @pl.when(b < meta_ref[nblk])
        def _():
            rows = lax.broadcasted_iota(jnp.int32, (bm, t), 0) + b * bm
            cmp0 = gd_ref[0:1, :] == rows
            cmp1 = gd_ref[1:2, :] == rows
            onehot = (cmp0 | cmp1).astype(jnp.bfloat16)
            wrow = jnp.sum(
                jnp.where(cmp0, wd_ref[0:1, :],
                          jnp.where(cmp1, wd_ref[1:2, :], 0.0)),
                axis=1, keepdims=True)
            x = jnp.dot(onehot, x_ref[...],
                        preferred_element_type=jnp.float32).astype(jnp.bfloat16)
            gu = jnp.dot(x, gu_ref[0].astype(jnp.bfloat16),
                         preferred_element_type=jnp.float32)
            gate = gu[:, :i]
            up = gu[:, i:]
            act = gate * jax.nn.sigmoid(gate) * up * wrow
            out_ref[...] = jnp.dot(act.astype(jnp.bfloat16),
                                   dp_ref[0].astype(jnp.bfloat16),
                                   preferred_element_type=jnp.float32)

    grid_spec = pltpu.PrefetchScalarGridSpec(
        num_scalar_prefetch=1,
        grid=(nblk,),
        in_specs=[
            pl.BlockSpec((2, t), lambda b, m: (0, 0)),
            pl.BlockSpec((2, t), lambda b, m: (0, 0)),
            pl.BlockSpec((t, h), lambda b, m: (0, 0)),
            pl.BlockSpec((1, h, i2), lambda b, m: (m[b], 0, 0)),
            pl.BlockSpec((1, i, h), lambda b, m: (m[b], 0, 0)),
        ],
        out_specs=pl.BlockSpec((bm, h), lambda b, m: (b, 0)),
    )
    return pl.pallas_call(
        body,
        grid_spec=grid_spec,
        out_shape=jax.ShapeDtypeStruct((pad, h), jnp.float32),
    )(meta, gd, wd, x_bf, gate_up_proj, down_proj)


def _sc_combine(part, g0, g1, t, h):
    tpw = t // _NW
    ch = 8
    nch = tpw // ch
    g0 = g0.reshape(_NW, nch, ch)
    g1 = g1.reshape(_NW, nch, ch)
    mesh = plsc.VectorSubcoreMesh(core_axis_name="c", subcore_axis_name="s")
    nvec = ch * (h // 16)
    cshift = 0
    hh = h // 16
    while (1 << cshift) < hh:
        cshift += 1

    @functools.partial(
        pl.kernel, mesh=mesh,
        out_type=jax.ShapeDtypeStruct((t, h), jnp.float32),
        scratch_types=[
            pltpu.VMEM((nch, ch), jnp.int32),
            pltpu.VMEM((nch, ch), jnp.int32),
            [pltpu.VMEM((ch, h), jnp.float32)] * 2,
            [pltpu.VMEM((ch, h), jnp.float32)] * 2,
            [pltpu.SemaphoreType.DMA] * 2,
            [pltpu.SemaphoreType.DMA] * 2,
            [pltpu.SemaphoreType.DMA] * 2,
        ],
    )
    def k(part_hbm, g0_hbm, g1_hbm, out_hbm, i0, i1, ba, bb, sga, sgb, sst):
        wid = lax.axis_index("s") * _NC + lax.axis_index("c")
        base = wid * tpw
        pltpu.sync_copy(g0_hbm.at[wid], i0)
        pltpu.sync_copy(g1_hbm.at[wid], i1)

        def g_start(j, p):
            pltpu.async_copy(part_hbm.at[i0.at[j]], ba[p], sga[p])
            pltpu.async_copy(part_hbm.at[i1.at[j]], bb[p], sgb[p])

        def g_wait(j, p):
            pltpu.make_async_copy(part_hbm.at[i0.at[j]], ba[p], sga[p]).wait()
            pltpu.make_async_copy(part_hbm.at[i1.at[j]], bb[p], sgb[p]).wait()

        def out_slice(j):
            return out_hbm.at[pl.ds(base + j * ch, ch)]

        def s_start(j, p):
            pltpu.async_copy(ba[p], out_slice(j), sst[p])

        def s_wait(j, p):
            pltpu.make_async_copy(ba[p], out_slice(j), sst[p]).wait()

        g_start(0, 0)
        for j in range(nch):
            p = j % 2
            q = (j + 1) % 2
            if j + 1 < nch:
                if j - 1 >= 0:
                    s_wait(j - 1, q)
                g_start(j + 1, q)
            g_wait(j, p)

            def add_body(tt, carry):
                r = lax.shift_right_logical(tt, cshift)
                c = pl.multiple_of(
                    lax.shift_left(lax.bitwise_and(tt, hh - 1), 4), 16)
                ba[p][r, pl.ds(c, 16)] = (ba[p][r, pl.ds(c, 16)]
                                          + bb[p][r, pl.ds(c, 16)])
                return carry

            lax.fori_loop(0, nvec, add_body, 0, unroll=4)
            s_start(j, p)
        s_wait(nch - 2, (nch - 2) % 2)
        s_wait(nch - 1, (nch - 1) % 2)

    return k(part, g0, g1)


def kernel(hidden_states, top_k_index, top_k_weights, gate_up_proj, down_proj):
    t, h = hidden_states.shape
    e = gate_up_proj.shape[0]
    k = top_k_index.shape[1]
    bm = 256
    n = t * k
    nblk = n // bm + e
    pad = nblk * bm

    tki_t = top_k_index.T.astype(jnp.int32)
    wd = top_k_weights.T.astype(jnp.float32)
    gd128, meta2, x_bf = _tc_meta(tki_t, hidden_states, e, bm, nblk, t, k)
    gd = gd128.reshape(k, t)
    meta = meta2.reshape(meta2.shape[0])
    part = _tc_ffn(x_bf, gd, wd, meta, gate_up_proj, down_proj, bm, nblk, pad)
    return _sc_combine(part, gd[0], gd[1], t, h)


# speedup vs baseline: 1.1113x; 1.0008x over previous
import functools

import jax
import jax.numpy as jnp
from jax import lax
from jax.experimental import pallas as pl
from jax.experimental.pallas import tpu as pltpu
from jax.experimental.pallas import tpu_sc as plsc

_NC, _NS = 2, 16
_NW = _NC * _NS


def _tc_meta(tki_t, hidden_states, num_experts, bm, nblk, t, k):
    n = t * k
    nseg = 32
    nrow = n // nseg
    ne = num_experts
    nc = nseg * ne
    h = hidden_states.shape[1]
    xblk = 512
    mrows = ((nblk + 1 + 7) // 8) * 8

    def outer_body(tki_ref, x_ref, gd_ref, meta_ref, xbf_ref):
        xbf_ref[...] = x_ref[...].astype(jnp.bfloat16)

        @pl.when(pl.program_id(0) == 0)
        def _():
            meta_body(tki_ref, gd_ref, meta_ref)

    def meta_body(tki_ref, gd_ref, meta_ref):
        fdot = functools.partial(jnp.dot, preferred_element_type=jnp.float32)

        def iot(shape, dim):
            return lax.broadcasted_iota(jnp.int32, shape, dim)

        ev_f = tki_ref[...].astype(jnp.float32)
        rep = (iot((nseg, nc), 1) // ne == iot((nseg, nc), 0)).astype(jnp.float32)
        ev_rep = fdot(ev_f, rep)
        ecol = (iot((nrow, nc), 1) % ne).astype(jnp.float32)
        oh = (ev_rep == ecol).astype(jnp.float32)
        tri = (iot((nrow, nrow), 0) >= iot((nrow, nrow), 1)).astype(jnp.float32)
        within = fdot(tri, oh)
        seg_flat = jnp.sum(oh, axis=0, keepdims=True)
        segm = ((iot((nc, nc), 0) // ne < iot((nc, nc), 1) // ne)
                & (iot((nc, nc), 0) % ne == iot((nc, nc), 1) % ne)
                ).astype(jnp.float32)
        prior = fdot(seg_flat, segm)
        rank_incl = within + prior
        csel = (iot((nc, ne), 0) % ne == iot((nc, ne), 1)).astype(jnp.float32)
        counts = fdot(seg_flat, csel)
        nblk_f = jnp.floor((counts + (bm - 1)) * (1.0 / bm))
        upper = (iot((ne, ne), 0) < iot((ne, ne), 1)).astype(jnp.float32)
        start_f = fdot(nblk_f, upper)
        start_rep = fdot(start_f, (iot((ne, nc), 1) % ne
                                   == iot((ne, nc), 0)).astype(jnp.float32))
        sel = oh * (start_rep * bm + rank_incl - 1.0)
        gsel = (iot((nc, nseg), 0) // ne == iot((nc, nseg), 1)).astype(jnp.float32)
        gd_ref[...] = jnp.dot(sel, gsel, preferred_element_type=jnp.float32,
                              precision=lax.Precision.HIGHEST).astype(jnp.int32)
        num_active = jnp.sum(nblk_f).astype(jnp.int32)
        start_i = start_f.astype(jnp.int32)
        br = lax.broadcasted_iota(jnp.int32, (mrows, ne), 0)
        be = jnp.sum((start_i <= br).astype(jnp.int32),
                     axis=1, keepdims=True) - 1
        be_last = jnp.sum((start_i <= num_active - 1).astype(jnp.int32)) - 1
        bcol = lax.broadcasted_iota(jnp.int32, (mrows, 1), 0)
        be = jnp.where(bcol < num_active, be, be_last)
        meta_ref[...] = jnp.where(bcol == nblk, num_active, be)

    return pl.pallas_call(
        outer_body,
        grid=(t // xblk,),
        in_specs=[pl.BlockSpec((nrow, nseg), lambda i: (0, 0)),
                  pl.BlockSpec((xblk, h), lambda i: (i, 0))],
        out_specs=[pl.BlockSpec((nrow, nseg), lambda i: (0, 0)),
                   pl.BlockSpec((mrows, 1), lambda i: (0, 0)),
                   pl.BlockSpec((xblk, h), lambda i: (i, 0))],
        out_shape=[jax.ShapeDtypeStruct((nrow, nseg), jnp.int32),
                   jax.ShapeDtypeStruct((mrows, 1), jnp.int32),
                   jax.ShapeDtypeStruct((t, h), jnp.bfloat16)],
    )(tki_t.reshape(nrow, nseg), hidden_states)


def _tc_ffn(x_bf, gd, wd, meta, gate_up_proj, down_proj, bm, nblk, pad):
    e, h, i2 = gate_up_proj.shape
    i = i2 // 2
    t = x_bf.shape[0]

    def body(meta_ref, gd_ref, wd_ref, x_ref, gu_ref, dp_ref, out_ref):
        b = pl.program_id(0)

        @pl.when(b < meta_ref[nblk])
        def _():
            rows = lax.broadcasted_iota(jnp.int32, (bm, t), 0) + b * bm
            cmp0 = gd_ref[0:1, :] == rows
            cmp1 = gd_ref[1:2, :] == rows
            onehot = (cmp0 | cmp1).astype(jnp.bfloat16)
            wrow = jnp.sum(
                jnp.where(cmp0, wd_ref[0:1, :],
                          jnp.where(cmp1, wd_ref[1:2, :], 0.0)),
                axis=1, keepdims=True)
            x = jnp.dot(onehot, x_ref[...],
                        preferred_element_type=jnp.float32).astype(jnp.bfloat16)
            gu = jnp.dot(x, gu_ref[0].astype(jnp.bfloat16),
                         preferred_element_type=jnp.float32)
            gate = gu[:, :i]
            up = gu[:, i:]
            act = gate * jax.nn.sigmoid(gate) * up * wrow
            out_ref[...] = jnp.dot(act.astype(jnp.bfloat16),
                                   dp_ref[0].astype(jnp.bfloat16),
                                   preferred_element_type=jnp.float32)

    grid_spec = pltpu.PrefetchScalarGridSpec(
        num_scalar_prefetch=1,
        grid=(nblk,),
        in_specs=[
            pl.BlockSpec((2, t), lambda b, m: (0, 0)),
            pl.BlockSpec((2, t), lambda b, m: (0, 0)),
            pl.BlockSpec((t, h), lambda b, m: (0, 0)),
            pl.BlockSpec((1, h, i2), lambda b, m: (m[b], 0, 0)),
            pl.BlockSpec((1, i, h), lambda b, m: (m[b], 0, 0)),
        ],
        out_specs=pl.BlockSpec((bm, h), lambda b, m: (b, 0)),
    )
    return pl.pallas_call(
        body,
        grid_spec=grid_spec,
        out_shape=jax.ShapeDtypeStruct((pad, h), jnp.float32),
    )(meta, gd, wd, x_bf, gate_up_proj, down_proj)


def _sc_combine(part, g0, g1, t, h):
    tpw = t // _NW
    ch = 8
    nch = tpw // ch
    g0 = g0.reshape(_NW, nch, ch)
    g1 = g1.reshape(_NW, nch, ch)
    mesh = plsc.VectorSubcoreMesh(core_axis_name="c", subcore_axis_name="s")
    nvec = ch * (h // 16)
    cshift = 0
    hh = h // 16
    while (1 << cshift) < hh:
        cshift += 1

    @functools.partial(
        pl.kernel, mesh=mesh,
        out_type=jax.ShapeDtypeStruct((t, h), jnp.float32),
        scratch_types=[
            pltpu.VMEM((nch, ch), jnp.int32),
            pltpu.VMEM((nch, ch), jnp.int32),
            [pltpu.VMEM((ch, h), jnp.float32)] * 2,
            [pltpu.VMEM((ch, h), jnp.float32)] * 2,
            [pltpu.SemaphoreType.DMA] * 2,
            [pltpu.SemaphoreType.DMA] * 2,
            [pltpu.SemaphoreType.DMA] * 2,
        ],
    )
    def k(part_hbm, g0_hbm, g1_hbm, out_hbm, i0, i1, ba, bb, sga, sgb, sst):
        wid = lax.axis_index("s") * _NC + lax.axis_index("c")
        base = wid * tpw
        pltpu.sync_copy(g0_hbm.at[wid], i0)
        pltpu.sync_copy(g1_hbm.at[wid], i1)

        def g_start(j, p):
            pltpu.async_copy(part_hbm.at[i0.at[j]], ba[p], sga[p])
            pltpu.async_copy(part_hbm.at[i1.at[j]], bb[p], sgb[p])

        def g_wait(j, p):
            pltpu.make_async_copy(part_hbm.at[i0.at[j]], ba[p], sga[p]).wait()
            pltpu.make_async_copy(part_hbm.at[i1.at[j]], bb[p], sgb[p]).wait()

        g_start(0, 0)
        for j in range(nch):
            p = j % 2
            q = (j + 1) % 2
            if j + 1 < nch:
                g_start(j + 1, q)
            g_wait(j, p)

            def add_body(tt, carry):
                r = lax.shift_right_logical(tt, cshift)
                c = pl.multiple_of(
                    lax.shift_left(lax.bitwise_and(tt, hh - 1), 4), 16)
                ba[p][r, pl.ds(c, 16)] = (ba[p][r, pl.ds(c, 16)]
                                          + bb[p][r, pl.ds(c, 16)])
                return carry

            lax.fori_loop(0, nvec, add_body, 0, unroll=4)
            pltpu.sync_copy(ba[p], out_hbm.at[pl.ds(base + j * ch, ch)])

    return k(part, g0, g1)


def kernel(hidden_states, top_k_index, top_k_weights, gate_up_proj, down_proj):
    t, h = hidden_states.shape
    e = gate_up_proj.shape[0]
    k = top_k_index.shape[1]
    bm = 256
    n = t * k
    nblk = n // bm + e
    pad = nblk * bm

    tki_t = top_k_index.T.astype(jnp.int32)
    wd = top_k_weights.T.astype(jnp.float32)
    gd128, meta2, x_bf = _tc_meta(tki_t, hidden_states, e, bm, nblk, t, k)
    gd = gd128.reshape(k, t)
    meta = meta2.reshape(meta2.shape[0])
    part = _tc_ffn(x_bf, gd, wd, meta, gate_up_proj, down_proj, bm, nblk, pad)
    return _sc_combine(part, gd[0], gd[1], t, h)


# speedup vs baseline: 1.1119x; 1.0005x over previous
import functools

import jax
import jax.numpy as jnp
from jax import lax
from jax.experimental import pallas as pl
from jax.experimental.pallas import tpu as pltpu
from jax.experimental.pallas import tpu_sc as plsc

_NC, _NS = 2, 16
_NW = _NC * _NS


def _tc_meta(tki_t, hidden_states, num_experts, bm, nblk, t, k):
    n = t * k
    nseg = 32
    nrow = n // nseg
    ne = num_experts
    nc = nseg * ne
    h = hidden_states.shape[1]
    xblk = 512
    mrows = ((nblk + 1 + 7) // 8) * 8

    def outer_body(tki_ref, x_ref, gd_ref, meta_ref, xbf_ref):
        xbf_ref[...] = x_ref[...].astype(jnp.bfloat16)

        @pl.when(pl.program_id(0) == 0)
        def _():
            meta_body(tki_ref, gd_ref, meta_ref)

    def meta_body(tki_ref, gd_ref, meta_ref):
        fdot = functools.partial(jnp.dot, preferred_element_type=jnp.float32)

        def iot(shape, dim):
            return lax.broadcasted_iota(jnp.int32, shape, dim)

        ev_f = tki_ref[...].astype(jnp.float32)
        rep = (iot((nseg, nc), 1) // ne == iot((nseg, nc), 0)).astype(jnp.float32)
        ev_rep = fdot(ev_f, rep)
        ecol = (iot((nrow, nc), 1) % ne).astype(jnp.float32)
        oh = (ev_rep == ecol).astype(jnp.float32)
        tri = (iot((nrow, nrow), 0) >= iot((nrow, nrow), 1)).astype(jnp.float32)
        within = fdot(tri, oh)
        seg_flat = jnp.sum(oh, axis=0, keepdims=True)
        segm = ((iot((nc, nc), 0) // ne < iot((nc, nc), 1) // ne)
                & (iot((nc, nc), 0) % ne == iot((nc, nc), 1) % ne)
                ).astype(jnp.float32)
        prior = fdot(seg_flat, segm)
        rank_incl = within + prior
        csel = (iot((nc, ne), 0) % ne == iot((nc, ne), 1)).astype(jnp.float32)
        counts = fdot(seg_flat, csel)
        nblk_f = jnp.floor((counts + (bm - 1)) * (1.0 / bm))
        upper = (iot((ne, ne), 0) < iot((ne, ne), 1)).astype(jnp.float32)
        start_f = fdot(nblk_f, upper)
        start_rep = fdot(start_f, (iot((ne, nc), 1) % ne
                                   == iot((ne, nc), 0)).astype(jnp.float32))
        sel = oh * (start_rep * bm + rank_incl - 1.0)
        gsel = (iot((nc, nseg), 0) // ne == iot((nc, nseg), 1)).astype(jnp.float32)
        gd_ref[...] = jnp.dot(sel, gsel, preferred_element_type=jnp.float32,
                              precision=lax.Precision.HIGHEST).astype(jnp.int32)
        num_active = jnp.sum(nblk_f).astype(jnp.int32)
        start_i = start_f.astype(jnp.int32)
        br = lax.broadcasted_iota(jnp.int32, (mrows, ne), 0)
        be = jnp.sum((start_i <= br).astype(jnp.int32),
                     axis=1, keepdims=True) - 1
        be_last = jnp.sum((start_i <= num_active - 1).astype(jnp.int32)) - 1
        bcol = lax.broadcasted_iota(jnp.int32, (mrows, 1), 0)
        be = jnp.where(bcol < num_active, be, be_last)
        meta_ref[...] = jnp.where(bcol == nblk, num_active, be)

    return pl.pallas_call(
        outer_body,
        grid=(t // xblk,),
        in_specs=[pl.BlockSpec((nrow, nseg), lambda i: (0, 0)),
                  pl.BlockSpec((xblk, h), lambda i: (i, 0))],
        out_specs=[pl.BlockSpec((nrow, nseg), lambda i: (0, 0)),
                   pl.BlockSpec((mrows, 1), lambda i: (0, 0)),
                   pl.BlockSpec((xblk, h), lambda i: (i, 0))],
        out_shape=[jax.ShapeDtypeStruct((nrow, nseg), jnp.int32),
                   jax.ShapeDtypeStruct((mrows, 1), jnp.int32),
                   jax.ShapeDtypeStruct((t, h), jnp.bfloat16)],
    )(tki_t.reshape(nrow, nseg), hidden_states)


def _tc_ffn(x_bf, gd, wd, meta, gate_up_proj, down_proj, bm, nblk, pad):
    e, h, i2 = gate_up_proj.shape
    i = i2 // 2
    t = x_bf.shape[0]

    def body(meta_ref, gd_ref, wd_ref, x_ref, gu_ref, dp_ref, out_ref):
        b = pl.program_id(0)

        @pl.when(b < meta_ref[nblk])
        def _():
            rows = lax.broadcasted_iota(jnp.int32, (bm, t), 0) + b * bm
            cmp0 = gd_ref[0:1, :] == rows
            cmp1 = gd_ref[1:2, :] == rows
            onehot = (cmp0 | cmp1).astype(jnp.bfloat16)
            wrow = jnp.sum(
                jnp.where(cmp0, wd_ref[0:1, :],
                          jnp.where(cmp1, wd_ref[1:2, :], 0.0)),
                axis=1, keepdims=True)
            x = jnp.dot(onehot, x_ref[...],
                        preferred_element_type=jnp.float32).astype(jnp.bfloat16)
            gu = jnp.dot(x, gu_ref[0].astype(jnp.bfloat16),
                         preferred_element_type=jnp.float32)
            gate = gu[:, :i]
            up = gu[:, i:]
            act = gate * jax.nn.sigmoid(gate) * up * wrow
            out_ref[...] = jnp.dot(act.astype(jnp.bfloat16),
                                   dp_ref[0].astype(jnp.bfloat16),
                                   preferred_element_type=jnp.float32)

    grid_spec = pltpu.PrefetchScalarGridSpec(
        num_scalar_prefetch=1,
        grid=(nblk,),
        in_specs=[
            pl.BlockSpec((2, t), lambda b, m: (0, 0)),
            pl.BlockSpec((2, t), lambda b, m: (0, 0)),
            pl.BlockSpec((t, h), lambda b, m: (0, 0)),
            pl.BlockSpec((1, h, i2), lambda b, m: (m[b], 0, 0)),
            pl.BlockSpec((1, i, h), lambda b, m: (m[b], 0, 0)),
        ],
        out_specs=pl.BlockSpec((bm, h), lambda b, m: (b, 0)),
    )
    return pl.pallas_call(
        body,
        grid_spec=grid_spec,
        out_shape=jax.ShapeDtypeStruct((pad, h), jnp.float32),
    )(meta, gd, wd, x_bf, gate_up_proj, down_proj)


def _sc_combine(part, g0, g1, t, h):
    tpw = t // _NW
    ch = 8
    nch = tpw // ch
    g0 = g0.reshape(_NW, nch, ch)
    g1 = g1.reshape(_NW, nch, ch)
    mesh = plsc.VectorSubcoreMesh(core_axis_name="c", subcore_axis_name="s")
    nvec = ch * (h // 16)
    cshift = 0
    hh = h // 16
    while (1 << cshift) < hh:
        cshift += 1

    @functools.partial(
        pl.kernel, mesh=mesh,
        out_type=jax.ShapeDtypeStruct((t, h), jnp.float32),
        scratch_types=[
            pltpu.VMEM((nch, ch), jnp.int32),
            pltpu.VMEM((nch, ch), jnp.int32),
            [pltpu.VMEM((ch, h), jnp.float32)] * 2,
            [pltpu.VMEM((ch, h), jnp.float32)] * 2,
            [pltpu.SemaphoreType.DMA] * 2,
            [pltpu.SemaphoreType.DMA] * 2,
        ],
    )
    def k(part_hbm, g0_hbm, g1_hbm, out_hbm, i0, i1, ba, bb, sga, sgb):
        wid = lax.axis_index("s") * _NC + lax.axis_index("c")
        base = wid * tpw
        pltpu.sync_copy(g0_hbm.at[wid], i0)
        pltpu.sync_copy(g1_hbm.at[wid], i1)

        def g_start(j, p):
            pltpu.async_copy(part_hbm.at[i0.at[j]], ba[p], sga[p])
            pltpu.async_copy(part_hbm.at[i1.at[j]], bb[p], sgb[p])

        def g_wait(j, p):
            pltpu.make_async_copy(part_hbm.at[i0.at[j]], ba[p], sga[p]).wait()
            pltpu.make_async_copy(part_hbm.at[i1.at[j]], bb[p], sgb[p]).wait()

        g_start(0, 0)
        for j in range(nch):
            p = j % 2
            q = (j + 1) % 2
            if j + 1 < nch:
                g_start(j + 1, q)
            g_wait(j, p)

            def add_body(tt, carry):
                r = lax.shift_right_logical(tt, cshift)
                c = pl.multiple_of(
                    lax.shift_left(lax.bitwise_and(tt, hh - 1), 4), 16)
                ba[p][r, pl.ds(c, 16)] = (ba[p][r, pl.ds(c, 16)]
                                          + bb[p][r, pl.ds(c, 16)])
                return carry

            lax.fori_loop(0, nvec, add_body, 0, unroll=4)
            pltpu.sync_copy(ba[p], out_hbm.at[pl.ds(base + j * ch, ch)])

    return k(part, g0, g1)


def kernel(hidden_states, top_k_index, top_k_weights, gate_up_proj, down_proj):
    t, h = hidden_states.shape
    e = gate_up_proj.shape[0]
    k = top_k_index.shape[1]
    bm = 256
    n = t * k
    nblk = n // bm + e
    pad = nblk * bm

    tki_t = top_k_index.T.astype(jnp.int32)
    wd = top_k_weights.T.astype(jnp.float32)
    gd128, meta2, x_bf = _tc_meta(tki_t, hidden_states, e, bm, nblk, t, k)
    gd = gd128.reshape(k, t)
    meta = meta2.reshape(meta2.shape[0])
    part = _tc_ffn(x_bf, gd, wd, meta, gate_up_proj, down_proj, bm, nblk, pad)
    return _sc_combine(part, gd[0], gd[1], t, h)
